# baseline (speedup 1.0000x reference)
import functools

import jax
import jax.numpy as jnp
import numpy as np
from jax import lax
from jax.experimental import pallas as pl
from jax.experimental.pallas import tpu as pltpu

N_NODES = 10000
N_EDGES = 160000
D_FEAT = 256
N_GRAPHS = 16
H = 4
C = 64
HC = H * C

_PREC = jax.lax.Precision.HIGHEST


def _proj_kernel(x_ref, w_ref, b_ref, o_ref):
    o_ref[...] = (
        jnp.dot(x_ref[...], w_ref[...], precision=_PREC,
                preferred_element_type=jnp.float32)
        + b_ref[...]
    )


def _projections(x, p):
    fi = x.shape[1]
    w4 = jnp.concatenate([p['Wq'], p['Wk'], p['Wv'], p['Ws']], axis=1)
    b4 = jnp.concatenate([p['bq'], p['bk'], p['bv'], p['bs']]).reshape(1, 4 * HC)
    blk = 1000
    grid = (N_NODES // blk,)
    return pl.pallas_call(
        _proj_kernel,
        grid=grid,
        in_specs=[
            pl.BlockSpec((blk, fi), lambda i: (i, 0)),
            pl.BlockSpec((fi, 4 * HC), lambda i: (0, 0)),
            pl.BlockSpec((1, 4 * HC), lambda i: (0, 0)),
        ],
        out_specs=pl.BlockSpec((blk, 4 * HC), lambda i: (i, 0)),
        out_shape=jax.ShapeDtypeStruct((N_NODES, 4 * HC), jnp.float32),
    )(x, w4, b4)


def _alpha_kernel(qi_ref, kg_ref, ea_ref, we_ref, o_ref):
    e = jnp.dot(ea_ref[...], we_ref[...], precision=_PREC,
                preferred_element_type=jnp.float32)
    kj = kg_ref[...] + e
    qi = qi_ref[...]
    scale = 1.0 / np.sqrt(C)
    parts = [
        jnp.sum(qi[:, h * C:(h + 1) * C] * kj[:, h * C:(h + 1) * C],
                axis=1, keepdims=True) * scale
        for h in range(H)
    ]
    o_ref[...] = jnp.concatenate(parts, axis=1)


def _edge_alpha(q_i, kv_j, ea_pad, we_pad):
    blk = 2000
    grid = (N_EDGES // blk,)
    return pl.pallas_call(
        _alpha_kernel,
        grid=grid,
        in_specs=[
            pl.BlockSpec((blk, HC), lambda i: (i, 0)),
            pl.BlockSpec((blk, HC), lambda i: (i, 0)),
            pl.BlockSpec((blk, 16), lambda i: (i, 0)),
            pl.BlockSpec((16, HC), lambda i: (0, 0)),
        ],
        out_specs=pl.BlockSpec((blk, H), lambda i: (i, 0)),
        out_shape=jax.ShapeDtypeStruct((N_EDGES, H), jnp.float32),
    )(q_i, kv_j, ea_pad, we_pad)


def _msg_kernel(al_ref, md_ref, vg_ref, ea_ref, we_ref, msg_ref, w_ref):
    w = jnp.exp(al_ref[...] - md_ref[...])
    e = jnp.dot(ea_ref[...], we_ref[...], precision=_PREC,
                preferred_element_type=jnp.float32)
    vj = vg_ref[...] + e
    parts = [vj[:, h * C:(h + 1) * C] * w[:, h:h + 1] for h in range(H)]
    msg_ref[...] = jnp.concatenate(parts, axis=1)
    w_ref[...] = w


def _edge_msg(alpha, m_dst, kv_j, ea_pad, we_pad):
    blk = 2000
    grid = (N_EDGES // blk,)
    return pl.pallas_call(
        _msg_kernel,
        grid=grid,
        in_specs=[
            pl.BlockSpec((blk, H), lambda i: (i, 0)),
            pl.BlockSpec((blk, H), lambda i: (i, 0)),
            pl.BlockSpec((blk, HC), lambda i: (i, 1)),
            pl.BlockSpec((blk, 16), lambda i: (i, 0)),
            pl.BlockSpec((16, HC), lambda i: (0, 0)),
        ],
        out_specs=[
            pl.BlockSpec((blk, HC), lambda i: (i, 0)),
            pl.BlockSpec((blk, H), lambda i: (i, 0)),
        ],
        out_shape=[
            jax.ShapeDtypeStruct((N_EDGES, HC), jnp.float32),
            jax.ShapeDtypeStruct((N_EDGES, H), jnp.float32),
        ],
    )(alpha, m_dst, kv_j, ea_pad, we_pad)


def _combine_kernel(ms_ref, dn_ref, sk_ref, w_ref, b_ref, bc_ref,
                    h_ref, pool_ref, acc_ref):
    pid = pl.program_id(0)
    nblk = pl.num_programs(0)

    @pl.when(pid == 0)
    def _init():
        acc_ref[...] = jnp.zeros_like(acc_ref)
        pool_ref[...] = jnp.full_like(pool_ref, -jnp.inf)

    dn = dn_ref[...]
    dn = jnp.where(dn > 0.0, dn, 1.0)
    ms = ms_ref[...]
    parts = [ms[:, h * C:(h + 1) * C] / dn[:, h:h + 1] for h in range(H)]
    node = jnp.concatenate(parts, axis=1) + sk_ref[...]
    hnew = (
        jnp.dot(node, w_ref[...], precision=_PREC,
                preferred_element_type=jnp.float32)
        + b_ref[...]
    )
    h_ref[...] = hnew

    bc = bc_ref[...]
    gids = lax.broadcasted_iota(jnp.int32, (bc.shape[0], N_GRAPHS), 1)
    oh_t = (bc == gids).astype(jnp.float32)
    ext = jnp.concatenate(
        [hnew, jnp.ones((bc.shape[0], 1), jnp.float32)], axis=1)
    psum = lax.dot_general(oh_t, ext, (((0,), (0,)), ((), ())),
                           precision=_PREC,
                           preferred_element_type=jnp.float32)
    acc_ref[...] += psum

    neg_inf = jnp.float32(-jnp.inf)
    gmax_rows = []
    for g in range(N_GRAPHS):
        masked = jnp.where(bc == g, hnew, neg_inf)
        gmax_rows.append(jnp.max(masked, axis=0, keepdims=True))
    bmax = jnp.concatenate(gmax_rows, axis=0)
    pool_ref[:, :C] = jnp.maximum(pool_ref[:, :C], bmax)

    @pl.when(pid == nblk - 1)
    def _fin():
        gmax = pool_ref[:, :C]
        gmax = jnp.where(jnp.isfinite(gmax), gmax, 0.0)
        counts = jnp.maximum(acc_ref[:, C:C + 1], 1.0)
        pool_ref[:, :C] = gmax
        pool_ref[:, C:] = acc_ref[:, :C] / counts


def _combine_pool(msg_sum, denom, skip, ht_w, ht_b, batch_col):
    blk = 1000
    grid = (N_NODES // blk,)
    return pl.pallas_call(
        _combine_kernel,
        grid=grid,
        in_specs=[
            pl.BlockSpec((blk, HC), lambda i: (i, 0)),
            pl.BlockSpec((blk, H), lambda i: (i, 0)),
            pl.BlockSpec((blk, HC), lambda i: (i, 0)),
            pl.BlockSpec((HC, C), lambda i: (0, 0)),
            pl.BlockSpec((1, C), lambda i: (0, 0)),
            pl.BlockSpec((blk, 1), lambda i: (i, 0)),
        ],
        out_specs=[
            pl.BlockSpec((blk, C), lambda i: (i, 0)),
            pl.BlockSpec((N_GRAPHS, 2 * C), lambda i: (0, 0)),
        ],
        out_shape=[
            jax.ShapeDtypeStruct((N_NODES, C), jnp.float32),
            jax.ShapeDtypeStruct((N_GRAPHS, 2 * C), jnp.float32),
        ],
        scratch_shapes=[pltpu.VMEM((N_GRAPHS, C + 1), jnp.float32)],
    )(msg_sum, denom, skip, ht_w, ht_b.reshape(1, C), batch_col)


def _mlp_kernel(pooled_ref, w1_ref, b1_ref, w2_ref, b2_ref, o_ref):
    z = jnp.maximum(
        jnp.dot(pooled_ref[...], w1_ref[...], precision=_PREC,
                preferred_element_type=jnp.float32) + b1_ref[...], 0.0)
    o_ref[...] = (
        jnp.dot(z, w2_ref[...], precision=_PREC,
                preferred_element_type=jnp.float32)
        + b2_ref[...]
    )


def _final_mlp(pooled, w1, b1, w2, b2):
    w2p = jnp.zeros((256, 128), jnp.float32).at[:, :2].set(w2)
    b2p = jnp.zeros((1, 128), jnp.float32).at[0, :2].set(b2)
    out = pl.pallas_call(
        _mlp_kernel,
        out_shape=jax.ShapeDtypeStruct((N_GRAPHS, 128), jnp.float32),
    )(pooled, w1, b1.reshape(1, -1), w2p, b2p)
    return out[:, :2]


def _layer(h, ea_pad, p, src, dst):
    proj = _projections(h, p)
    q = proj[:, :HC]
    kv = proj[:, HC:3 * HC]
    skip = proj[:, 3 * HC:]

    q_i = jnp.take(q, dst, axis=0)
    kv_j = jnp.take(kv, src, axis=0)

    we_pad = jnp.pad(p['We'], ((0, 5), (0, 0)))
    alpha = _edge_alpha(q_i, kv_j, ea_pad, we_pad)
    m = jax.ops.segment_max(alpha, dst, num_segments=N_NODES)
    m_dst = jnp.take(m, dst, axis=0)
    msg, w = _edge_msg(alpha, m_dst, kv_j, ea_pad, we_pad)
    denom = jax.ops.segment_sum(w, dst, num_segments=N_NODES)
    msg_sum = jax.ops.segment_sum(msg, dst, num_segments=N_NODES)
    return msg_sum, denom, skip


def kernel(x, edge_attr, params, edge_index, batch_index):
    src = edge_index[0]
    dst = edge_index[1]
    ea_pad = jnp.pad(edge_attr, ((0, 0), (0, 5)))
    batch_col = batch_index.reshape(N_NODES, 1)

    h = x
    pooled = jnp.zeros((N_GRAPHS, 2 * C), jnp.float32)
    for i in range(5):
        p = params['conv%d' % (i + 1)]
        msg_sum, denom, skip = _layer(h, ea_pad, p, src, dst)
        ht = params['ht%d' % (i + 1)]
        h, pool_i = _combine_pool(msg_sum, denom, skip, ht['W'], ht['b'],
                                  batch_col)
        pooled = pooled + pool_i
    return _final_mlp(pooled, params['l1W'], params['l1b'],
                      params['l2W'], params['l2b'])


# speedup vs baseline: 1.0327x; 1.0327x over previous
import functools

import jax
import jax.numpy as jnp
import numpy as np
from jax import lax
from jax.experimental import pallas as pl
from jax.experimental.pallas import tpu as pltpu

N_NODES = 10000
N_EDGES = 160000
D_FEAT = 256
N_GRAPHS = 16
H = 4
C = 64
HC = H * C

_PREC = jax.lax.Precision.HIGHEST


def _proj_kernel(x_ref, w_ref, b_ref, o_ref):
    o_ref[...] = (
        jnp.dot(x_ref[...], w_ref[...], precision=_PREC,
                preferred_element_type=jnp.float32)
        + b_ref[...]
    )


def _projections(x, p):
    fi = x.shape[1]
    w4 = jnp.concatenate([p['Wq'], p['Wk'], p['Wv'], p['Ws']], axis=1)
    b4 = jnp.concatenate([p['bq'], p['bk'], p['bv'], p['bs']]).reshape(1, 4 * HC)
    blk = 1000
    grid = (N_NODES // blk,)
    return pl.pallas_call(
        _proj_kernel,
        grid=grid,
        in_specs=[
            pl.BlockSpec((blk, fi), lambda i: (i, 0)),
            pl.BlockSpec((fi, 4 * HC), lambda i: (0, 0)),
            pl.BlockSpec((1, 4 * HC), lambda i: (0, 0)),
        ],
        out_specs=pl.BlockSpec((blk, 4 * HC), lambda i: (i, 0)),
        out_shape=jax.ShapeDtypeStruct((N_NODES, 4 * HC), jnp.float32),
    )(x, w4, b4)


def _alpha_kernel(qi_ref, kg_ref, ea_ref, we_ref, o_ref):
    e = jnp.dot(ea_ref[...], we_ref[...], precision=_PREC,
                preferred_element_type=jnp.float32)
    kj = kg_ref[...] + e
    qi = qi_ref[...]
    scale = 1.0 / np.sqrt(C)
    parts = [
        jnp.sum(qi[:, h * C:(h + 1) * C] * kj[:, h * C:(h + 1) * C],
                axis=1, keepdims=True) * scale
        for h in range(H)
    ]
    o_ref[...] = jnp.concatenate(parts, axis=1)


def _edge_alpha(q_i, kv_j, ea_pad, we_pad):
    n_e = q_i.shape[0]
    blk = 4000
    grid = (n_e // blk,)
    return pl.pallas_call(
        _alpha_kernel,
        grid=grid,
        in_specs=[
            pl.BlockSpec((blk, HC), lambda i: (i, 0)),
            pl.BlockSpec((blk, HC), lambda i: (i, 0)),
            pl.BlockSpec((blk, 16), lambda i: (i, 0)),
            pl.BlockSpec((16, HC), lambda i: (0, 0)),
        ],
        out_specs=pl.BlockSpec((blk, H), lambda i: (i, 0)),
        out_shape=jax.ShapeDtypeStruct((n_e, H), jnp.float32),
    )(q_i, kv_j, ea_pad, we_pad)


def _msg_kernel(al_ref, md_ref, vg_ref, ea_ref, we_ref, msg_ref, w_ref):
    w = jnp.exp(al_ref[...] - md_ref[...])
    e = jnp.dot(ea_ref[...], we_ref[...], precision=_PREC,
                preferred_element_type=jnp.float32)
    vj = vg_ref[...] + e
    parts = [vj[:, h * C:(h + 1) * C] * w[:, h:h + 1] for h in range(H)]
    msg_ref[...] = jnp.concatenate(parts, axis=1)
    w_ref[...] = w


def _edge_msg(alpha, m_dst, kv_j, ea_pad, we_pad):
    n_e = alpha.shape[0]
    blk = 4000
    grid = (n_e // blk,)
    return pl.pallas_call(
        _msg_kernel,
        grid=grid,
        in_specs=[
            pl.BlockSpec((blk, H), lambda i: (i, 0)),
            pl.BlockSpec((blk, H), lambda i: (i, 0)),
            pl.BlockSpec((blk, HC), lambda i: (i, 1)),
            pl.BlockSpec((blk, 16), lambda i: (i, 0)),
            pl.BlockSpec((16, HC), lambda i: (0, 0)),
        ],
        out_specs=[
            pl.BlockSpec((blk, HC), lambda i: (i, 0)),
            pl.BlockSpec((blk, H), lambda i: (i, 0)),
        ],
        out_shape=[
            jax.ShapeDtypeStruct((n_e, HC), jnp.float32),
            jax.ShapeDtypeStruct((n_e, H), jnp.float32),
        ],
    )(alpha, m_dst, kv_j, ea_pad, we_pad)


def _combine_kernel(ms_ref, dn_ref, sk_ref, w_ref, b_ref, bc_ref,
                    h_ref, pool_ref, acc_ref):
    pid = pl.program_id(0)
    nblk = pl.num_programs(0)

    @pl.when(pid == 0)
    def _init():
        acc_ref[...] = jnp.zeros_like(acc_ref)
        pool_ref[...] = jnp.full_like(pool_ref, -jnp.inf)

    dn = dn_ref[...]
    dn = jnp.where(dn > 0.0, dn, 1.0)
    ms = ms_ref[...]
    parts = [ms[:, h * C:(h + 1) * C] / dn[:, h:h + 1] for h in range(H)]
    node = jnp.concatenate(parts, axis=1) + sk_ref[...]
    hnew = (
        jnp.dot(node, w_ref[...], precision=_PREC,
                preferred_element_type=jnp.float32)
        + b_ref[...]
    )
    h_ref[...] = hnew

    bc = bc_ref[...]
    gids = lax.broadcasted_iota(jnp.int32, (bc.shape[0], N_GRAPHS), 1)
    oh_t = (bc == gids).astype(jnp.float32)
    ext = jnp.concatenate(
        [hnew, jnp.ones((bc.shape[0], 1), jnp.float32)], axis=1)
    psum = lax.dot_general(oh_t, ext, (((0,), (0,)), ((), ())),
                           precision=_PREC,
                           preferred_element_type=jnp.float32)
    acc_ref[...] += psum

    neg_inf = jnp.float32(-jnp.inf)
    gmax_rows = []
    for g in range(N_GRAPHS):
        masked = jnp.where(bc == g, hnew, neg_inf)
        gmax_rows.append(jnp.max(masked, axis=0, keepdims=True))
    bmax = jnp.concatenate(gmax_rows, axis=0)
    pool_ref[:, :C] = jnp.maximum(pool_ref[:, :C], bmax)

    @pl.when(pid == nblk - 1)
    def _fin():
        gmax = pool_ref[:, :C]
        gmax = jnp.where(jnp.isfinite(gmax), gmax, 0.0)
        counts = jnp.maximum(acc_ref[:, C:C + 1], 1.0)
        pool_ref[:, :C] = gmax
        pool_ref[:, C:] = acc_ref[:, :C] / counts


def _combine_pool(msg_sum, denom, skip, ht_w, ht_b, batch_col):
    blk = 1000
    grid = (N_NODES // blk,)
    return pl.pallas_call(
        _combine_kernel,
        grid=grid,
        in_specs=[
            pl.BlockSpec((blk, HC), lambda i: (i, 0)),
            pl.BlockSpec((blk, H), lambda i: (i, 0)),
            pl.BlockSpec((blk, HC), lambda i: (i, 0)),
            pl.BlockSpec((HC, C), lambda i: (0, 0)),
            pl.BlockSpec((1, C), lambda i: (0, 0)),
            pl.BlockSpec((blk, 1), lambda i: (i, 0)),
        ],
        out_specs=[
            pl.BlockSpec((blk, C), lambda i: (i, 0)),
            pl.BlockSpec((N_GRAPHS, 2 * C), lambda i: (0, 0)),
        ],
        out_shape=[
            jax.ShapeDtypeStruct((N_NODES, C), jnp.float32),
            jax.ShapeDtypeStruct((N_GRAPHS, 2 * C), jnp.float32),
        ],
        scratch_shapes=[pltpu.VMEM((N_GRAPHS, C + 1), jnp.float32)],
    )(msg_sum, denom, skip, ht_w, ht_b.reshape(1, C), batch_col)


def _mlp_kernel(pooled_ref, w1_ref, b1_ref, w2_ref, b2_ref, o_ref):
    z = jnp.maximum(
        jnp.dot(pooled_ref[...], w1_ref[...], precision=_PREC,
                preferred_element_type=jnp.float32) + b1_ref[...], 0.0)
    o_ref[...] = (
        jnp.dot(z, w2_ref[...], precision=_PREC,
                preferred_element_type=jnp.float32)
        + b2_ref[...]
    )


def _final_mlp(pooled, w1, b1, w2, b2):
    w2p = jnp.zeros((256, 128), jnp.float32).at[:, :2].set(w2)
    b2p = jnp.zeros((1, 128), jnp.float32).at[0, :2].set(b2)
    out = pl.pallas_call(
        _mlp_kernel,
        out_shape=jax.ShapeDtypeStruct((N_GRAPHS, 128), jnp.float32),
    )(pooled, w1, b1.reshape(1, -1), w2p, b2p)
    return out[:, :2]


_NCHUNK = 2


def _layer(h, ea_pad, p, src, dst):
    proj = _projections(h, p)
    q = proj[:, :HC]
    kv = proj[:, HC:3 * HC]
    skip = proj[:, 3 * HC:]

    we_pad = jnp.pad(p['We'], ((0, 5), (0, 0)))
    ec = N_EDGES // _NCHUNK

    kv_js, alphas = [], []
    m = None
    for c in range(_NCHUNK):
        sl = slice(c * ec, (c + 1) * ec)
        q_i = jnp.take(q, dst[sl], axis=0)
        kv_j = jnp.take(kv, src[sl], axis=0)
        kv_js.append(kv_j)
        alpha = _edge_alpha(q_i, kv_j, ea_pad[sl], we_pad)
        alphas.append(alpha)
        m_c = jax.ops.segment_max(alpha, dst[sl], num_segments=N_NODES)
        m = m_c if m is None else jnp.maximum(m, m_c)

    denom = jnp.zeros((N_NODES, H), jnp.float32)
    msg_sum = jnp.zeros((N_NODES, HC), jnp.float32)
    for c in range(_NCHUNK):
        sl = slice(c * ec, (c + 1) * ec)
        m_dst = jnp.take(m, dst[sl], axis=0)
        msg, w = _edge_msg(alphas[c], m_dst, kv_js[c], ea_pad[sl], we_pad)
        denom = denom + jax.ops.segment_sum(w, dst[sl], num_segments=N_NODES)
        msg_sum = msg_sum + jax.ops.segment_sum(msg, dst[sl],
                                                num_segments=N_NODES)
    return msg_sum, denom, skip


def kernel(x, edge_attr, params, edge_index, batch_index):
    src = edge_index[0]
    dst = edge_index[1]
    ea_pad = jnp.pad(edge_attr, ((0, 0), (0, 5)))
    batch_col = batch_index.reshape(N_NODES, 1)

    h = x
    pooled = jnp.zeros((N_GRAPHS, 2 * C), jnp.float32)
    for i in range(5):
        p = params['conv%d' % (i + 1)]
        msg_sum, denom, skip = _layer(h, ea_pad, p, src, dst)
        ht = params['ht%d' % (i + 1)]
        h, pool_i = _combine_pool(msg_sum, denom, skip, ht['W'], ht['b'],
                                  batch_col)
        pooled = pooled + pool_i
    return _final_mlp(pooled, params['l1W'], params['l1b'],
                      params['l2W'], params['l2b'])


# speedup vs baseline: 1.1892x; 1.1515x over previous
import functools

import jax
import jax.numpy as jnp
import numpy as np
from jax import lax
from jax.experimental import pallas as pl
from jax.experimental.pallas import tpu as pltpu

N_NODES = 10000
N_EDGES = 160000
D_FEAT = 256
N_GRAPHS = 16
H = 4
C = 64
HC = H * C

_PREC = jax.lax.Precision.DEFAULT


def _proj_kernel(x_ref, w_ref, b_ref, o_ref):
    o_ref[...] = (
        jnp.dot(x_ref[...], w_ref[...], precision=_PREC,
                preferred_element_type=jnp.float32)
        + b_ref[...]
    )


def _projections(x, p):
    fi = x.shape[1]
    w4 = jnp.concatenate([p['Wq'], p['Wk'], p['Wv'], p['Ws']], axis=1)
    b4 = jnp.concatenate([p['bq'], p['bk'], p['bv'], p['bs']]).reshape(1, 4 * HC)
    blk = 1000
    grid = (N_NODES // blk,)
    return pl.pallas_call(
        _proj_kernel,
        grid=grid,
        in_specs=[
            pl.BlockSpec((blk, fi), lambda i: (i, 0)),
            pl.BlockSpec((fi, 4 * HC), lambda i: (0, 0)),
            pl.BlockSpec((1, 4 * HC), lambda i: (0, 0)),
        ],
        out_specs=pl.BlockSpec((blk, 4 * HC), lambda i: (i, 0)),
        out_shape=jax.ShapeDtypeStruct((N_NODES, 4 * HC), jnp.float32),
    )(x, w4, b4)


def _alpha_kernel(qi_ref, kg_ref, ea_ref, we_ref, o_ref):
    e = jnp.dot(ea_ref[...], we_ref[...], precision=_PREC,
                preferred_element_type=jnp.float32)
    kj = kg_ref[...] + e
    qi = qi_ref[...]
    scale = 1.0 / np.sqrt(C)
    parts = [
        jnp.sum(qi[:, h * C:(h + 1) * C] * kj[:, h * C:(h + 1) * C],
                axis=1, keepdims=True) * scale
        for h in range(H)
    ]
    o_ref[...] = jnp.concatenate(parts, axis=1)


def _edge_alpha(q_i, kv_j, ea_pad, we_pad):
    n_e = q_i.shape[0]
    blk = 4000
    grid = (n_e // blk,)
    return pl.pallas_call(
        _alpha_kernel,
        grid=grid,
        in_specs=[
            pl.BlockSpec((blk, HC), lambda i: (i, 0)),
            pl.BlockSpec((blk, HC), lambda i: (i, 0)),
            pl.BlockSpec((blk, 16), lambda i: (i, 0)),
            pl.BlockSpec((16, HC), lambda i: (0, 0)),
        ],
        out_specs=pl.BlockSpec((blk, H), lambda i: (i, 0)),
        out_shape=jax.ShapeDtypeStruct((n_e, H), jnp.float32),
    )(q_i, kv_j, ea_pad, we_pad)


def _msg_kernel(al_ref, md_ref, vg_ref, ea_ref, we_ref, msg_ref, w_ref):
    w = jnp.exp(al_ref[...] - md_ref[...])
    e = jnp.dot(ea_ref[...], we_ref[...], precision=_PREC,
                preferred_element_type=jnp.float32)
    vj = vg_ref[...] + e
    parts = [vj[:, h * C:(h + 1) * C] * w[:, h:h + 1] for h in range(H)]
    msg_ref[...] = jnp.concatenate(parts, axis=1)
    w_ref[...] = w


def _edge_msg(alpha, m_dst, kv_j, ea_pad, we_pad):
    n_e = alpha.shape[0]
    blk = 4000
    grid = (n_e // blk,)
    return pl.pallas_call(
        _msg_kernel,
        grid=grid,
        in_specs=[
            pl.BlockSpec((blk, H), lambda i: (i, 0)),
            pl.BlockSpec((blk, H), lambda i: (i, 0)),
            pl.BlockSpec((blk, HC), lambda i: (i, 1)),
            pl.BlockSpec((blk, 16), lambda i: (i, 0)),
            pl.BlockSpec((16, HC), lambda i: (0, 0)),
        ],
        out_specs=[
            pl.BlockSpec((blk, HC), lambda i: (i, 0)),
            pl.BlockSpec((blk, H), lambda i: (i, 0)),
        ],
        out_shape=[
            jax.ShapeDtypeStruct((n_e, HC), jnp.float32),
            jax.ShapeDtypeStruct((n_e, H), jnp.float32),
        ],
    )(alpha, m_dst, kv_j, ea_pad, we_pad)


def _combine_kernel(ms_ref, dn_ref, sk_ref, w_ref, b_ref, bc_ref,
                    h_ref, pool_ref, acc_ref):
    pid = pl.program_id(0)
    nblk = pl.num_programs(0)

    @pl.when(pid == 0)
    def _init():
        acc_ref[...] = jnp.zeros_like(acc_ref)
        pool_ref[...] = jnp.full_like(pool_ref, -jnp.inf)

    dn = dn_ref[...]
    dn = jnp.where(dn > 0.0, dn, 1.0)
    ms = ms_ref[...]
    parts = [ms[:, h * C:(h + 1) * C] / dn[:, h:h + 1] for h in range(H)]
    node = jnp.concatenate(parts, axis=1) + sk_ref[...]
    hnew = (
        jnp.dot(node, w_ref[...], precision=_PREC,
                preferred_element_type=jnp.float32)
        + b_ref[...]
    )
    h_ref[...] = hnew

    bc = bc_ref[...]
    gids = lax.broadcasted_iota(jnp.int32, (bc.shape[0], N_GRAPHS), 1)
    oh_t = (bc == gids).astype(jnp.float32)
    ext = jnp.concatenate(
        [hnew, jnp.ones((bc.shape[0], 1), jnp.float32)], axis=1)
    psum = lax.dot_general(oh_t, ext, (((0,), (0,)), ((), ())),
                           precision=_PREC,
                           preferred_element_type=jnp.float32)
    acc_ref[...] += psum

    neg_inf = jnp.float32(-jnp.inf)
    gmax_rows = []
    for g in range(N_GRAPHS):
        masked = jnp.where(bc == g, hnew, neg_inf)
        gmax_rows.append(jnp.max(masked, axis=0, keepdims=True))
    bmax = jnp.concatenate(gmax_rows, axis=0)
    pool_ref[:, :C] = jnp.maximum(pool_ref[:, :C], bmax)

    @pl.when(pid == nblk - 1)
    def _fin():
        gmax = pool_ref[:, :C]
        gmax = jnp.where(jnp.isfinite(gmax), gmax, 0.0)
        counts = jnp.maximum(acc_ref[:, C:C + 1], 1.0)
        pool_ref[:, :C] = gmax
        pool_ref[:, C:] = acc_ref[:, :C] / counts


def _combine_pool(msg_sum, denom, skip, ht_w, ht_b, batch_col):
    blk = 1000
    grid = (N_NODES // blk,)
    return pl.pallas_call(
        _combine_kernel,
        grid=grid,
        in_specs=[
            pl.BlockSpec((blk, HC), lambda i: (i, 0)),
            pl.BlockSpec((blk, H), lambda i: (i, 0)),
            pl.BlockSpec((blk, HC), lambda i: (i, 0)),
            pl.BlockSpec((HC, C), lambda i: (0, 0)),
            pl.BlockSpec((1, C), lambda i: (0, 0)),
            pl.BlockSpec((blk, 1), lambda i: (i, 0)),
        ],
        out_specs=[
            pl.BlockSpec((blk, C), lambda i: (i, 0)),
            pl.BlockSpec((N_GRAPHS, 2 * C), lambda i: (0, 0)),
        ],
        out_shape=[
            jax.ShapeDtypeStruct((N_NODES, C), jnp.float32),
            jax.ShapeDtypeStruct((N_GRAPHS, 2 * C), jnp.float32),
        ],
        scratch_shapes=[pltpu.VMEM((N_GRAPHS, C + 1), jnp.float32)],
    )(msg_sum, denom, skip, ht_w, ht_b.reshape(1, C), batch_col)


def _mlp_kernel(pooled_ref, w1_ref, b1_ref, w2_ref, b2_ref, o_ref):
    z = jnp.maximum(
        jnp.dot(pooled_ref[...], w1_ref[...], precision=_PREC,
                preferred_element_type=jnp.float32) + b1_ref[...], 0.0)
    o_ref[...] = (
        jnp.dot(z, w2_ref[...], precision=_PREC,
                preferred_element_type=jnp.float32)
        + b2_ref[...]
    )


def _final_mlp(pooled, w1, b1, w2, b2):
    w2p = jnp.zeros((256, 128), jnp.float32).at[:, :2].set(w2)
    b2p = jnp.zeros((1, 128), jnp.float32).at[0, :2].set(b2)
    out = pl.pallas_call(
        _mlp_kernel,
        out_shape=jax.ShapeDtypeStruct((N_GRAPHS, 128), jnp.float32),
    )(pooled, w1, b1.reshape(1, -1), w2p, b2p)
    return out[:, :2]


_NCHUNK = 2


def _layer(h, ea_pad, p, src, dst):
    proj = _projections(h, p)
    q = proj[:, :HC]
    kv = proj[:, HC:3 * HC]
    skip = proj[:, 3 * HC:]

    we_pad = jnp.pad(p['We'], ((0, 5), (0, 0)))
    ec = N_EDGES // _NCHUNK

    kv_js, alphas = [], []
    m = None
    for c in range(_NCHUNK):
        sl = slice(c * ec, (c + 1) * ec)
        q_i = jnp.take(q, dst[sl], axis=0)
        kv_j = jnp.take(kv, src[sl], axis=0)
        kv_js.append(kv_j)
        alpha = _edge_alpha(q_i, kv_j, ea_pad[sl], we_pad)
        alphas.append(alpha)
        m_c = jax.ops.segment_max(alpha, dst[sl], num_segments=N_NODES)
        m = m_c if m is None else jnp.maximum(m, m_c)

    denom = jnp.zeros((N_NODES, H), jnp.float32)
    msg_sum = jnp.zeros((N_NODES, HC), jnp.float32)
    for c in range(_NCHUNK):
        sl = slice(c * ec, (c + 1) * ec)
        m_dst = jnp.take(m, dst[sl], axis=0)
        msg, w = _edge_msg(alphas[c], m_dst, kv_js[c], ea_pad[sl], we_pad)
        denom = denom + jax.ops.segment_sum(w, dst[sl], num_segments=N_NODES)
        msg_sum = msg_sum + jax.ops.segment_sum(msg, dst[sl],
                                                num_segments=N_NODES)
    return msg_sum, denom, skip


def kernel(x, edge_attr, params, edge_index, batch_index):
    src = edge_index[0]
    dst = edge_index[1]
    ea_pad = jnp.pad(edge_attr, ((0, 0), (0, 5)))
    batch_col = batch_index.reshape(N_NODES, 1)

    h = x
    pooled = jnp.zeros((N_GRAPHS, 2 * C), jnp.float32)
    for i in range(5):
        p = params['conv%d' % (i + 1)]
        msg_sum, denom, skip = _layer(h, ea_pad, p, src, dst)
        ht = params['ht%d' % (i + 1)]
        h, pool_i = _combine_pool(msg_sum, denom, skip, ht['W'], ht['b'],
                                  batch_col)
        pooled = pooled + pool_i
    return _final_mlp(pooled, params['l1W'], params['l1b'],
                      params['l2W'], params['l2b'])
